# baseline (speedup 1.0000x reference)
import numpy as np

import jax
import jax.numpy as jnp
from jax import lax
from jax.experimental import pallas as pl
from jax.experimental.pallas import tpu as pltpu

N = 32768
ROWS = 256
COLS = 128
CHUNKS = 32
K = 32
SPIKE_THR = 2.0


def _xor_rows(a, j):
    rows = a.shape[0]
    if j >= 8:
        blocks = [a[k * 8:(k + 1) * 8, :] for k in range(rows // 8)]
        perm = [blocks[((k * 8) ^ j) // 8] for k in range(rows // 8)]
        return jnp.concatenate(perm, axis=0)
    i = lax.broadcasted_iota(jnp.int32, (rows, COLS), 0)
    bit = (i & j) != 0
    up = pltpu.roll(a, rows - j, axis=0)
    dn = pltpu.roll(a, j, axis=0)
    return jnp.where(bit, dn, up)


def _xor_perm(a, j):
    return _xor_rows(a, j)


def _rev_rows(a):
    rows = a.shape[0]
    a = jnp.concatenate([a[k * 8:(k + 1) * 8, :]
                         for k in range(rows // 8 - 1, -1, -1)], axis=0)
    for j in (4, 2, 1):
        a = _xor_perm(a, j)
    return a


def _merge_ce(a, j):
    rows = a.shape[0]
    i = lax.broadcasted_iota(jnp.int32, (rows, COLS), 0)
    bit = (i & j) != 0
    if j >= 8:
        partner = _xor_rows(a, j)
        return jnp.where(bit, jnp.minimum(a, partner),
                         jnp.maximum(a, partner))
    up = pltpu.roll(a, rows - j, axis=0)
    dn = pltpu.roll(a, j, axis=0)
    return jnp.where(bit, jnp.minimum(a, dn), jnp.maximum(a, up))


def _ce(a, j, k):
    rows = a.shape[0]
    i = lax.broadcasted_iota(jnp.int32, (rows, COLS), 0)
    bit = (i & j) != 0
    if j >= 8:
        partner = _xor_rows(a, j)
    else:
        up = pltpu.roll(a, rows - j, axis=0)
        dn = pltpu.roll(a, j, axis=0)
        partner = jnp.where(bit, dn, up)
    if k is None:
        keepmax = jnp.logical_not(bit)
    else:
        keepmax = ((i & k) == 0) == jnp.logical_not(bit)
    return jnp.where(keepmax, jnp.maximum(a, partner),
                     jnp.minimum(a, partner))


def _desc_sort16(a):
    k = 2
    while k <= 16:
        j = k // 2
        while j >= 1:
            a = _ce(a, j, k)
            j //= 2
        k *= 2
    return a


def _merge_top32(a, b):
    c = jnp.maximum(a, _rev_rows(b))
    for j in (16, 8, 4, 2, 1):
        c = _merge_ce(c, j)
    return c


def _tc_body(x_ref, y_ref, w_ref):
    X = x_ref[...]
    neg = jnp.float32(-jnp.inf)
    big = jnp.int32(1 << 30)

    m1 = jnp.full((8, COLS), neg, jnp.float32)
    m2 = jnp.full((8, COLS), neg, jnp.float32)
    for c in range(CHUNKS):
        ch = X[8 * c:8 * (c + 1), :]
        nm1 = jnp.maximum(m1, ch)
        m2 = jnp.maximum(m2, jnp.minimum(m1, ch))
        m1 = nm1
    layers = jnp.concatenate([m1, m2], axis=0)

    w16 = _desc_sort16(layers)

    w = jnp.concatenate(
        [w16, _rev_rows(pltpu.roll(w16, 1, axis=1))], axis=0)
    for j in (16, 8, 4, 2, 1):
        w = _merge_ce(w, j)

    for l in range(1, 7):
        w = _merge_top32(w, pltpu.roll(w, 1 << l, axis=1))

    trow = w[31:32, :]
    t8 = jnp.broadcast_to(trow, (8, COLS))

    cgt_v = jnp.zeros((8, COLS), jnp.int32)
    ceq_v = jnp.zeros((8, COLS), jnp.int32)
    one = jnp.ones((8, COLS), jnp.int32)
    zero = jnp.zeros((8, COLS), jnp.int32)
    for c in range(CHUNKS):
        ch = X[8 * c:8 * (c + 1), :]
        cgt_v += jnp.where(ch > t8, one, zero)
        ceq_v += jnp.where(ch == t8, one, zero)
    cgt = jnp.sum(cgt_v)
    ceq = jnp.sum(ceq_v)
    clay = jnp.sum((layers > jnp.broadcast_to(trow, (16, COLS)))
                   .astype(jnp.int32))
    exact = (clay == cgt) & (ceq <= K - cgt)

    @pl.when(exact)
    def _fast():
        t256 = jnp.broadcast_to(trow, (ROWS, COLS))
        win = (X >= t256) & (X > SPIKE_THR)
        y_ref[...] = jnp.where(win, jnp.float32(1.0), jnp.float32(0.0))

    @pl.when(jnp.logical_not(exact))
    def _exact_fallback():
        posf = (lax.broadcasted_iota(jnp.int32, (ROWS, COLS), 0) * COLS
                + lax.broadcasted_iota(jnp.int32, (ROWS, COLS), 1))
        w_ref[...] = X
        y_ref[...] = jnp.zeros((ROWS, COLS), jnp.float32)

        def f_step(_, __):
            wv = w_ref[...]
            m = jnp.max(wv)
            p = jnp.min(jnp.where(wv == m, posf, big))
            hit = posf == p
            y_ref[...] = jnp.where(hit & (m > SPIKE_THR),
                                   jnp.float32(1.0), y_ref[...])
            w_ref[...] = jnp.where(hit, neg, wv)
            return 0

        lax.fori_loop(0, K, f_step, 0)


def kernel(x):
    y = pl.pallas_call(
        _tc_body,
        out_shape=jax.ShapeDtypeStruct((ROWS, COLS), jnp.float32),
        scratch_shapes=[pltpu.VMEM((ROWS, COLS), jnp.float32)],
    )(x.reshape(ROWS, COLS))
    return y.reshape(N)


# speedup vs baseline: 1.0106x; 1.0106x over previous
import numpy as np

import jax
import jax.numpy as jnp
from jax import lax
from jax.experimental import pallas as pl
from jax.experimental.pallas import tpu as pltpu

N = 32768
ROWS = 256
COLS = 128
CHUNKS = 32
K = 32
SPIKE_THR = 2.0


def _xor_rows(a, j):
    rows = a.shape[0]
    if j >= 8:
        blocks = [a[k * 8:(k + 1) * 8, :] for k in range(rows // 8)]
        perm = [blocks[((k * 8) ^ j) // 8] for k in range(rows // 8)]
        return jnp.concatenate(perm, axis=0)
    i = lax.broadcasted_iota(jnp.int32, (rows, COLS), 0)
    bit = (i & j) != 0
    up = pltpu.roll(a, rows - j, axis=0)
    dn = pltpu.roll(a, j, axis=0)
    return jnp.where(bit, dn, up)


def _xor_perm(a, j):
    return _xor_rows(a, j)


def _rev_rows(a):
    rows = a.shape[0]
    a = jnp.concatenate([a[k * 8:(k + 1) * 8, :]
                         for k in range(rows // 8 - 1, -1, -1)], axis=0)
    for j in (4, 2, 1):
        a = _xor_perm(a, j)
    return a


def _merge_ce(a, j):
    rows = a.shape[0]
    i = lax.broadcasted_iota(jnp.int32, (rows, COLS), 0)
    bit = (i & j) != 0
    if j >= 8:
        partner = _xor_rows(a, j)
        return jnp.where(bit, jnp.minimum(a, partner),
                         jnp.maximum(a, partner))
    up = pltpu.roll(a, rows - j, axis=0)
    dn = pltpu.roll(a, j, axis=0)
    return jnp.where(bit, jnp.minimum(a, dn), jnp.maximum(a, up))


def _ce(a, j, k):
    rows = a.shape[0]
    i = lax.broadcasted_iota(jnp.int32, (rows, COLS), 0)
    bit = (i & j) != 0
    if j >= 8:
        partner = _xor_rows(a, j)
    else:
        up = pltpu.roll(a, rows - j, axis=0)
        dn = pltpu.roll(a, j, axis=0)
        partner = jnp.where(bit, dn, up)
    if k is None:
        keepmax = jnp.logical_not(bit)
    else:
        keepmax = ((i & k) == 0) == jnp.logical_not(bit)
    return jnp.where(keepmax, jnp.maximum(a, partner),
                     jnp.minimum(a, partner))


def _desc_sort16(a):
    k = 2
    while k <= 16:
        j = k // 2
        while j >= 1:
            a = _ce(a, j, k)
            j //= 2
        k *= 2
    return a


def _merge_top32(a, b):
    c = jnp.maximum(a, _rev_rows(b))
    for j in (16, 8, 4, 2, 1):
        c = _merge_ce(c, j)
    return c


def _tc_body(x_ref, y_ref, w_ref):
    X = x_ref[...]
    neg = jnp.float32(-jnp.inf)
    big = jnp.int32(1 << 30)

    parts = []
    for g in range(4):
        p1 = X[8 * (8 * g):8 * (8 * g) + 8, :]
        p2 = jnp.full((8, COLS), neg, jnp.float32)
        for c in range(8 * g + 1, 8 * g + 8):
            ch = X[8 * c:8 * (c + 1), :]
            np1 = jnp.maximum(p1, ch)
            p2 = jnp.maximum(p2, jnp.minimum(p1, ch))
            p1 = np1
        parts.append((p1, p2))

    def top2_merge(a, b):
        a1, a2 = a
        b1, b2 = b
        m1 = jnp.maximum(a1, b1)
        m2 = jnp.maximum(jnp.minimum(a1, b1),
                         jnp.where(a1 >= b1, a2, b2))
        return m1, m2

    ab = top2_merge(parts[0], parts[1])
    cd = top2_merge(parts[2], parts[3])
    m1, m2 = top2_merge(ab, cd)
    layers = jnp.concatenate([m1, m2], axis=0)

    w16 = _desc_sort16(layers)

    w = jnp.concatenate(
        [w16, _rev_rows(pltpu.roll(w16, 1, axis=1))], axis=0)
    for j in (16, 8, 4, 2, 1):
        w = _merge_ce(w, j)

    for l in range(1, 7):
        w = _merge_top32(w, pltpu.roll(w, 1 << l, axis=1))

    trow = w[31:32, :]
    t8 = jnp.broadcast_to(trow, (8, COLS))

    one = jnp.ones((8, COLS), jnp.int32)
    zero = jnp.zeros((8, COLS), jnp.int32)
    cgt_acc = []
    ceq_acc = []
    for g in range(4):
        cgt_v = jnp.zeros((8, COLS), jnp.int32)
        ceq_v = jnp.zeros((8, COLS), jnp.int32)
        for c in range(8 * g, 8 * g + 8):
            ch = X[8 * c:8 * (c + 1), :]
            cgt_v += jnp.where(ch > t8, one, zero)
            ceq_v += jnp.where(ch == t8, one, zero)
        cgt_acc.append(cgt_v)
        ceq_acc.append(ceq_v)
    cgt = jnp.sum((cgt_acc[0] + cgt_acc[1]) + (cgt_acc[2] + cgt_acc[3]))
    ceq = jnp.sum((ceq_acc[0] + ceq_acc[1]) + (ceq_acc[2] + ceq_acc[3]))
    clay = jnp.sum((layers > jnp.broadcast_to(trow, (16, COLS)))
                   .astype(jnp.int32))
    exact = (clay == cgt) & (ceq <= K - cgt)

    @pl.when(exact)
    def _fast():
        t256 = jnp.broadcast_to(trow, (ROWS, COLS))
        win = (X >= t256) & (X > SPIKE_THR)
        y_ref[...] = jnp.where(win, jnp.float32(1.0), jnp.float32(0.0))

    @pl.when(jnp.logical_not(exact))
    def _exact_fallback():
        posf = (lax.broadcasted_iota(jnp.int32, (ROWS, COLS), 0) * COLS
                + lax.broadcasted_iota(jnp.int32, (ROWS, COLS), 1))
        w_ref[...] = X
        y_ref[...] = jnp.zeros((ROWS, COLS), jnp.float32)

        def f_step(_, __):
            wv = w_ref[...]
            m = jnp.max(wv)
            p = jnp.min(jnp.where(wv == m, posf, big))
            hit = posf == p
            y_ref[...] = jnp.where(hit & (m > SPIKE_THR),
                                   jnp.float32(1.0), y_ref[...])
            w_ref[...] = jnp.where(hit, neg, wv)
            return 0

        lax.fori_loop(0, K, f_step, 0)


def kernel(x):
    y = pl.pallas_call(
        _tc_body,
        out_shape=jax.ShapeDtypeStruct((ROWS, COLS), jnp.float32),
        scratch_shapes=[pltpu.VMEM((ROWS, COLS), jnp.float32)],
    )(x.reshape(ROWS, COLS))
    return y.reshape(N)
